# baseline (speedup 1.0000x reference)
import jax
import jax.numpy as jnp
from jax import lax
from jax.experimental import pallas as pl

_BR = 256
_K = 5
_W = 128


def _ce(a, b):
    return jnp.minimum(a, b), jnp.maximum(a, b)


def _sort4(a, b, c, d):
    a, b = _ce(a, b)
    c, d = _ce(c, d)
    a, c = _ce(a, c)
    b, d = _ce(b, d)
    b, c = _ce(b, c)
    return a, b, c, d


def _merge44_keep5(a1, a2, a3, a4, b1, b2, b3, b4):
    z1, t1 = _ce(a1, b1)
    z2, t2 = _ce(a2, b2)
    z3, t3 = _ce(a3, b3)
    z4 = jnp.minimum(a4, b4)
    z2, c = _ce(z2, t1)
    z3, c = _ce(z3, c)
    z4, z5 = _ce(z4, c)
    z3, c = _ce(z3, t2)
    z4, c = _ce(z4, c)
    z5 = jnp.minimum(z5, c)
    z4, c = _ce(z4, t3)
    z5 = jnp.minimum(z5, c)
    return z1, z2, z3, z4, z5


def _merge55_keep5(p, q):
    p1, p2, p3, p4, p5 = p
    q1, q2, q3, q4, q5 = q
    s1, h1 = _ce(p1, q1)
    s2, h2 = _ce(p2, q2)
    s3, h3 = _ce(p3, q3)
    s4, h4 = _ce(p4, q4)
    s5 = jnp.minimum(p5, q5)
    s2, c = _ce(s2, h1)
    s3, c = _ce(s3, c)
    s4, c = _ce(s4, c)
    s5 = jnp.minimum(s5, c)
    s3, c = _ce(s3, h2)
    s4, c = _ce(s4, c)
    s5 = jnp.minimum(s5, c)
    s4, c = _ce(s4, h3)
    s5 = jnp.minimum(s5, c)
    s5 = jnp.minimum(s5, h4)
    return s1, s2, s3, s4, s5


def _cost_body(q_ref, l_ref, t_ref, c_ref, o_ref):
    nt = t_ref.shape[1]
    br = q_ref.shape[0]
    qx = q_ref[:, 0:1]
    qy = q_ref[:, 1:2]
    tx = t_ref[0:1, :]
    ty = t_ref[1:2, :]
    dx = qx - tx
    dy = qy - ty
    s2 = dx * dx + dy * dy
    d = jnp.sqrt(s2)

    nslice = nt // _W
    vs = [d[:, c * _W:(c + 1) * _W] for c in range(nslice)]
    if nslice == 16:
        g = [_sort4(*vs[i * 4:(i + 1) * 4]) for i in range(4)]
        p = _merge44_keep5(*g[0], *g[1])
        q = _merge44_keep5(*g[2], *g[3])
        ms = list(_merge55_keep5(p, q))
    else:
        ms = [jnp.full((br, _W), jnp.inf, jnp.float32) for _ in range(_K)]
        for c in range(nslice):
            new = vs[c]
            for k in range(_K):
                lo = jnp.minimum(ms[k], new)
                new = jnp.maximum(ms[k], new)
                ms[k] = lo

    iota = lax.broadcasted_iota(jnp.int32, (br, _W), 1)
    total = jnp.zeros((br, 1), jnp.float32)
    for i in range(_K):
        m = jnp.min(ms[0], axis=1, keepdims=True)
        total = total + m
        if i < _K - 1:
            pos = jnp.min(jnp.where(ms[0] <= m, iota, _W), axis=1,
                          keepdims=True)
            sel = iota == pos
            for k in range(_K - 1 - i):
                ms[k] = jnp.where(sel, ms[k + 1], ms[k])
    delta = total / jnp.float32(_K)

    p0 = jax.nn.sigmoid(l_ref[:, 0:1] - l_ref[:, 1:2])
    tsel = c_ref[0:1, :] > 0.5
    cls_cost = jnp.where(tsel, 1.0 - p0, p0)

    w = jnp.exp2(s2 * (-1.4426950408889634 / 50.0))
    cost_point = jnp.where(d < delta, d * w, d)
    o_ref[:, :] = cost_point - cls_cost


@jax.jit
def kernel(pred_logits, pred_points, tgt_points, tgt_ids):
    bs, nq, _ = pred_logits.shape
    nt = tgt_points.shape[0]
    nq_flat = bs * nq
    q = pred_points.reshape(nq_flat, 2)
    logits = pred_logits.reshape(nq_flat, 2)
    t_t = tgt_points.T
    cls = tgt_ids.astype(jnp.float32).reshape(1, nt)

    out = pl.pallas_call(
        _cost_body,
        grid=(nq_flat // _BR,),
        in_specs=[
            pl.BlockSpec((_BR, 2), lambda i: (i, 0)),
            pl.BlockSpec((_BR, 2), lambda i: (i, 0)),
            pl.BlockSpec((2, nt), lambda i: (0, 0)),
            pl.BlockSpec((1, nt), lambda i: (0, 0)),
        ],
        out_specs=pl.BlockSpec((_BR, nt), lambda i: (i, 0)),
        out_shape=jax.ShapeDtypeStruct((nq_flat, nt), jnp.float32),
    )(q, logits, t_t, cls)
    return out.reshape(bs, nq, nt)


# speedup vs baseline: 1.1441x; 1.1441x over previous
import jax
import jax.numpy as jnp
from jax import lax
from jax.experimental import pallas as pl

_BR = 512
_K = 5
_W = 128


def _ce(a, b):
    return jnp.minimum(a, b), jnp.maximum(a, b)


def _sort4(a, b, c, d):
    a, b = _ce(a, b)
    c, d = _ce(c, d)
    a, c = _ce(a, c)
    b, d = _ce(b, d)
    b, c = _ce(b, c)
    return a, b, c, d


def _merge44_keep5(a1, a2, a3, a4, b1, b2, b3, b4):
    z1, t1 = _ce(a1, b1)
    z2, t2 = _ce(a2, b2)
    z3, t3 = _ce(a3, b3)
    z4 = jnp.minimum(a4, b4)
    z2, c = _ce(z2, t1)
    z3, c = _ce(z3, c)
    z4, z5 = _ce(z4, c)
    z3, c = _ce(z3, t2)
    z4, c = _ce(z4, c)
    z5 = jnp.minimum(z5, c)
    z4, c = _ce(z4, t3)
    z5 = jnp.minimum(z5, c)
    return z1, z2, z3, z4, z5


def _merge55_keep5(p, q):
    p1, p2, p3, p4, p5 = p
    q1, q2, q3, q4, q5 = q
    s1, h1 = _ce(p1, q1)
    s2, h2 = _ce(p2, q2)
    s3, h3 = _ce(p3, q3)
    s4, h4 = _ce(p4, q4)
    s5 = jnp.minimum(p5, q5)
    s2, c = _ce(s2, h1)
    s3, c = _ce(s3, c)
    s4, c = _ce(s4, c)
    s5 = jnp.minimum(s5, c)
    s3, c = _ce(s3, h2)
    s4, c = _ce(s4, c)
    s5 = jnp.minimum(s5, c)
    s4, c = _ce(s4, h3)
    s5 = jnp.minimum(s5, c)
    s5 = jnp.minimum(s5, h4)
    return s1, s2, s3, s4, s5


def _cost_body(q_ref, l_ref, t_ref, c_ref, o_ref):
    nt = t_ref.shape[1]
    br = q_ref.shape[0]
    qx = q_ref[:, 0:1]
    qy = q_ref[:, 1:2]
    tx = t_ref[0:1, :]
    ty = t_ref[1:2, :]
    dx = qx - tx
    dy = qy - ty
    s2 = dx * dx + dy * dy
    d = jnp.sqrt(s2)

    nslice = nt // _W
    vs = [d[:, c * _W:(c + 1) * _W] for c in range(nslice)]
    if nslice == 16:
        g = [_sort4(*vs[i * 4:(i + 1) * 4]) for i in range(4)]
        p = _merge44_keep5(*g[0], *g[1])
        q = _merge44_keep5(*g[2], *g[3])
        ms = list(_merge55_keep5(p, q))
    else:
        ms = [jnp.full((br, _W), jnp.inf, jnp.float32) for _ in range(_K)]
        for c in range(nslice):
            new = vs[c]
            for k in range(_K):
                lo = jnp.minimum(ms[k], new)
                new = jnp.maximum(ms[k], new)
                ms[k] = lo

    iota = lax.broadcasted_iota(jnp.int32, (br, _W), 1)
    total = jnp.zeros((br, 1), jnp.float32)
    for i in range(_K):
        m = jnp.min(ms[0], axis=1, keepdims=True)
        total = total + m
        if i < _K - 1:
            pos = jnp.min(jnp.where(ms[0] <= m, iota, _W), axis=1,
                          keepdims=True)
            sel = iota == pos
            for k in range(_K - 1 - i):
                ms[k] = jnp.where(sel, ms[k + 1], ms[k])
    delta = total / jnp.float32(_K)

    p0 = jax.nn.sigmoid(l_ref[:, 0:1] - l_ref[:, 1:2])
    tsel = c_ref[0:1, :] > 0.5
    cls_cost = jnp.where(tsel, 1.0 - p0, p0)

    w = jnp.exp2(s2 * (-1.4426950408889634 / 50.0))
    cost_point = jnp.where(d < delta, d * w, d)
    o_ref[:, :] = cost_point - cls_cost


@jax.jit
def kernel(pred_logits, pred_points, tgt_points, tgt_ids):
    bs, nq, _ = pred_logits.shape
    nt = tgt_points.shape[0]
    nq_flat = bs * nq
    q = pred_points.reshape(nq_flat, 2)
    logits = pred_logits.reshape(nq_flat, 2)
    t_t = tgt_points.T
    cls = tgt_ids.astype(jnp.float32).reshape(1, nt)

    out = pl.pallas_call(
        _cost_body,
        grid=(nq_flat // _BR,),
        in_specs=[
            pl.BlockSpec((_BR, 2), lambda i: (i, 0)),
            pl.BlockSpec((_BR, 2), lambda i: (i, 0)),
            pl.BlockSpec((2, nt), lambda i: (0, 0)),
            pl.BlockSpec((1, nt), lambda i: (0, 0)),
        ],
        out_specs=pl.BlockSpec((_BR, nt), lambda i: (i, 0)),
        out_shape=jax.ShapeDtypeStruct((nq_flat, nt), jnp.float32),
    )(q, logits, t_t, cls)
    return out.reshape(bs, nq, nt)
